# baseline (speedup 1.0000x reference)
import functools

import jax
import jax.numpy as jnp
from jax import lax
from jax.experimental import pallas as pl
from jax.experimental.pallas import tpu as pltpu
from jax.experimental.pallas import tpu_sc as plsc

N = 10000
E = 320000
D = 128
G = 128
L = 3
BN_SCALE = float(1.0 / (1.0 + 1e-5) ** 0.5)


NB = 2000
NSTEPS = N // NB

ER = E // 128
EBR = ER


def _prep_body(x_ref, atom_ref, vne_ref, bond_ref, hin0_ref, tbl_ref):
    step = pl.program_id(0)
    iota128 = lax.broadcasted_iota(jnp.int32, (NB, 128), 1)
    acc = jnp.zeros((NB, D), jnp.float32)
    for f in range(9):
        col = x_ref[:, pl.ds(f, 1)]
        oh = (col == iota128).astype(jnp.float32)
        acc = acc + lax.dot(oh, atom_ref[f], preferred_element_type=jnp.float32)
    hin0_ref[...] = acc + vne_ref[...]

    @pl.when(step == 0)
    def _():
        b0 = bond_ref[0]
        b1 = bond_ref[1]
        b2 = bond_ref[2]
        b12 = (b1[:, None, :] + b2[None, :, :]).reshape(256, D)
        for i in range(16):
            tbl_ref[pl.ds(i * 256, 256), :] = b12 + b0[i, :][None, :]


def _fidx_body(ea_ref, fidx_ref):
    a = ea_ref[...]
    fidx_ref[...] = a[0] * 256 + a[1] * 16 + a[2]


def _layer_body(hin_ref, agg_ref, batch_ref, w1_ref, b1_ref, w2_ref, b2_ref,
                eps_ref, h_ref, *maybe_pooled, last):
    pooled_ref = maybe_pooled[0] if maybe_pooled else None
    step = pl.program_id(0)
    hin = hin_ref[...]
    pre = (1.0 + eps_ref[0, 0]) * hin + agg_ref[0] + agg_ref[1]
    t = lax.dot_general(pre, w1_ref[...], (((1,), (1,)), ((), ())),
                        preferred_element_type=jnp.float32)
    t = jnp.maximum((t + b1_ref[...]) * BN_SCALE, 0.0)
    h = lax.dot_general(t, w2_ref[...], (((1,), (1,)), ((), ())),
                        preferred_element_type=jnp.float32)
    h = (h + b2_ref[...]) * BN_SCALE
    if not last:
        h = jnp.maximum(h, 0.0)
    h_ref[...] = h
    if not last:
        oh = (batch_ref[...] == lax.broadcasted_iota(jnp.int32, (NB, G), 1)
              ).astype(jnp.float32)
        p = lax.dot_general(oh, hin, (((0,), (0,)), ((), ())),
                            preferred_element_type=jnp.float32)

        @pl.when(step == 0)
        def _():
            pooled_ref[...] = p

        @pl.when(step > 0)
        def _():
            pooled_ref[...] = pooled_ref[...] + p


def _vn_body(pooled_ref, vncur_ref, w1_ref, b1_ref, w2_ref, b2_ref,
             batch_ref, h_ref, vnnext_ref, hinnext_ref, vscr_ref):
    step = pl.program_id(0)

    @pl.when(step == 0)
    def _():
        pt = pooled_ref[...] + vncur_ref[...]
        t2 = lax.dot_general(pt, w1_ref[...], (((1,), (1,)), ((), ())),
                             preferred_element_type=jnp.float32)
        t2 = jnp.maximum((t2 + b1_ref[...]) * BN_SCALE, 0.0)
        v = lax.dot_general(t2, w2_ref[...], (((1,), (1,)), ((), ())),
                            preferred_element_type=jnp.float32)
        v = jnp.maximum((v + b2_ref[...]) * BN_SCALE, 0.0)
        vnnext_ref[...] = v
        vscr_ref[...] = v

    oh = (batch_ref[...] == lax.broadcasted_iota(jnp.int32, (NB, G), 1)
          ).astype(jnp.float32)
    hinnext_ref[...] = h_ref[...] + lax.dot(oh, vscr_ref[...],
                                            preferred_element_type=jnp.float32)


NC = 2
NS = 16
NW = NC * NS
EC = 128
CPW = 80
EPAD = NW * CPW * EC
NCH = EPAD // EC
NPAD = N + 8
RPT = 624


def _sc_agg_body(hin, tbl, gidx2, dst2, zeros, out,
                 big, gib, dib, sgm,
                 agg_sh):
    cid = lax.axis_index("c")
    sid = lax.axis_index("s")
    wid = sid * NC + cid

    lo = sid * RPT
    pltpu.sync_copy(zeros.at[pl.ds(lo, RPT)], agg_sh.at[pl.ds(lo, RPT)])

    @pl.when(sid == NS - 1)
    def _():
        pltpu.sync_copy(zeros.at[pl.ds(NS * RPT, NPAD - NS * RPT)],
                        agg_sh.at[pl.ds(NS * RPT, NPAD - NS * RPT)])

    plsc.subcore_barrier()

    base = wid * CPW

    def group(t, carry):
        g8 = base + 8 * t
        pltpu.sync_copy(gidx2.at[pl.ds(2 * g8, 16)], gib)
        pltpu.sync_copy(dst2.at[pl.ds(g8, 8)], dib)

        for i in range(8):
            pltpu.async_copy(hin.at[gib.at[2 * i]],
                             big.at[pl.ds(0, EC)], sgm)
            pltpu.async_copy(tbl.at[gib.at[2 * i + 1]],
                             big.at[pl.ds(EC, EC)], sgm)
            pltpu.make_async_copy(hin.at[gib.at[2 * i]],
                                  big.at[pl.ds(0, EC)], sgm).wait()
            pltpu.make_async_copy(tbl.at[gib.at[2 * i + 1]],
                                  big.at[pl.ds(EC, EC)], sgm).wait()

            def erow(e, c2):
                for k in range(8):
                    s_ = pl.ds(k * 16, 16)
                    big[e, s_] = jnp.maximum(big[e, s_] + big[EC + e, s_], 0.0)
                return c2

            lax.fori_loop(0, EC, erow, 0, unroll=False)
            pltpu.sync_copy(big.at[pl.ds(0, EC)], agg_sh.at[dib.at[i]],
                            add=True)
        return carry

    lax.fori_loop(0, CPW // 8, group, 0, unroll=False)

    plsc.subcore_barrier()
    pltpu.sync_copy(agg_sh.at[pl.ds(lo, RPT)],
                    out.at[pl.ds(cid * N + lo, RPT)])

    @pl.when(sid == NS - 1)
    def _():
        pltpu.sync_copy(agg_sh.at[pl.ds(NS * RPT, N - NS * RPT)],
                        out.at[pl.ds(cid * N + NS * RPT, N - NS * RPT)])


def _sc_agg(hin, tbl, gidx2, dst2, zeros):
    mesh = plsc.VectorSubcoreMesh(core_axis_name="c", subcore_axis_name="s")
    f = pl.kernel(
        _sc_agg_body,
        out_type=jax.ShapeDtypeStruct((2 * N, D), jnp.float32),
        mesh=mesh,
        scratch_types=[
            pltpu.VMEM((2 * EC, D), jnp.float32),
            pltpu.VMEM((16, EC), jnp.int32),
            pltpu.VMEM((8, EC), jnp.int32),
            pltpu.SemaphoreType.DMA,
            pltpu.VMEM_SHARED((NPAD, D), jnp.float32),
        ],
    )
    return f(hin, tbl, gidx2, dst2, zeros)


def _prep(x, atom_emb, vn_emb, bond_emb):
    return pl.pallas_call(
        _prep_body,
        grid=(NSTEPS,),
        in_specs=[
            pl.BlockSpec((NB, 9), lambda i: (i, 0)),
            pl.BlockSpec((9, 128, D), lambda i: (0, 0, 0)),
            pl.BlockSpec((1, D), lambda i: (0, 0)),
            pl.BlockSpec((3, 16, D), lambda i: (0, 0, 0)),
        ],
        out_specs=[
            pl.BlockSpec((NB, D), lambda i: (i, 0)),
            pl.BlockSpec((4096, D), lambda i: (0, 0)),
        ],
        out_shape=[
            jax.ShapeDtypeStruct((N, D), jnp.float32),
            jax.ShapeDtypeStruct((4096, D), jnp.float32),
        ],
    )(x, atom_emb, vn_emb, bond_emb)


def _fidx(eat):
    return pl.pallas_call(
        _fidx_body,
        grid=(ER // EBR,),
        in_specs=[pl.BlockSpec((3, EBR, 128), lambda i: (0, i, 0))],
        out_specs=pl.BlockSpec((EBR, 128), lambda i: (i, 0)),
        out_shape=jax.ShapeDtypeStruct((ER, 128), jnp.int32),
    )(eat)


def _layer(hin, agg2, batch2, w1, b1, w2, b2, eps, last):
    out_specs = [pl.BlockSpec((NB, D), lambda i: (i, 0))]
    out_shape = [jax.ShapeDtypeStruct((N, D), jnp.float32)]
    if not last:
        out_specs.append(pl.BlockSpec((G, D), lambda i: (0, 0)))
        out_shape.append(jax.ShapeDtypeStruct((G, D), jnp.float32))
    res = pl.pallas_call(
        functools.partial(_layer_body, last=last),
        grid=(NSTEPS,),
        in_specs=[
            pl.BlockSpec((NB, D), lambda i: (i, 0)),
            pl.BlockSpec((2, NB, D), lambda i: (0, i, 0)),
            pl.BlockSpec((NB, 1), lambda i: (i, 0)),
            pl.BlockSpec((D, D), lambda i: (0, 0)),
            pl.BlockSpec((1, D), lambda i: (0, 0)),
            pl.BlockSpec((D, D), lambda i: (0, 0)),
            pl.BlockSpec((1, D), lambda i: (0, 0)),
            pl.BlockSpec(memory_space=pltpu.SMEM),
        ],
        out_specs=out_specs,
        out_shape=out_shape,
    )(hin, agg2, batch2, w1, b1, w2, b2, eps)
    if last:
        return res[0], None
    return res[0], res[1]


def _vn_apply(pooled, vncur, w1, b1, w2, b2, batch2, h):
    return pl.pallas_call(
        _vn_body,
        grid=(NSTEPS,),
        in_specs=[
            pl.BlockSpec((G, D), lambda i: (0, 0)),
            pl.BlockSpec((G, D), lambda i: (0, 0)),
            pl.BlockSpec((D, D), lambda i: (0, 0)),
            pl.BlockSpec((1, D), lambda i: (0, 0)),
            pl.BlockSpec((D, D), lambda i: (0, 0)),
            pl.BlockSpec((1, D), lambda i: (0, 0)),
            pl.BlockSpec((NB, 1), lambda i: (i, 0)),
            pl.BlockSpec((NB, D), lambda i: (i, 0)),
        ],
        out_specs=[
            pl.BlockSpec((G, D), lambda i: (0, 0)),
            pl.BlockSpec((NB, D), lambda i: (i, 0)),
        ],
        out_shape=[
            jax.ShapeDtypeStruct((G, D), jnp.float32),
            jax.ShapeDtypeStruct((N, D), jnp.float32),
        ],
        scratch_shapes=[pltpu.VMEM((G, D), jnp.float32)],
    )(pooled, vncur, w1, b1, w2, b2, batch2, h)


def kernel(x, edge_index, edge_attr, batch, atom_emb, bond_emb, vn_emb,
           gin_lin1_W, gin_lin1_b, gin_lin2_W, gin_lin2_b, gin_eps,
           vn_lin1_W, vn_lin1_b, vn_lin2_W, vn_lin2_b):
    pad = EPAD - E
    src1 = jnp.concatenate([edge_index[0], jnp.zeros((pad,), jnp.int32)])
    dst1 = jnp.concatenate([edge_index[1], jnp.full((pad,), N, jnp.int32)])
    eat = edge_attr.T.reshape(3, ER, 128)
    batch2 = batch.reshape(N, 1)
    zeros = jnp.zeros((NPAD, D), jnp.float32)

    hin0, tbl = _prep(x, atom_emb, vn_emb, bond_emb)
    fidx1 = jnp.concatenate([_fidx(eat).reshape(E),
                             jnp.zeros((pad,), jnp.int32)])
    gidx2 = jnp.concatenate([src1.reshape(NCH, 1, EC),
                             fidx1.reshape(NCH, 1, EC)],
                            axis=1).reshape(NCH * 2, EC)
    dst2 = dst1.reshape(NCH, EC)

    vn = jnp.broadcast_to(vn_emb, (G, D))
    hin = hin0
    h_list = [hin0]
    for layer in range(L):
        last = layer == L - 1
        agg2 = _sc_agg(hin, tbl, gidx2, dst2, zeros).reshape(2, N, D)
        eps = gin_eps[layer].reshape(1, 1)
        h, pooled = _layer(hin, agg2, batch2,
                           gin_lin1_W[layer], gin_lin1_b[layer].reshape(1, D),
                           gin_lin2_W[layer], gin_lin2_b[layer].reshape(1, D),
                           eps, last)
        if last:
            h_list.append(h)
        else:
            vn, hin = _vn_apply(pooled, vn,
                                vn_lin1_W[layer], vn_lin1_b[layer].reshape(1, D),
                                vn_lin2_W[layer], vn_lin2_b[layer].reshape(1, D),
                                batch2, h)
            h_list.append(hin)
    return jnp.stack(h_list)


# speedup vs baseline: 1.1650x; 1.1650x over previous
import functools

import jax
import jax.numpy as jnp
from jax import lax
from jax.experimental import pallas as pl
from jax.experimental.pallas import tpu as pltpu
from jax.experimental.pallas import tpu_sc as plsc

N = 10000
E = 320000
D = 128
G = 128
L = 3
BN_SCALE = float(1.0 / (1.0 + 1e-5) ** 0.5)


NB = 2000
NSTEPS = N // NB

ER = E // 128
EBR = ER


def _prep_body(x_ref, atom_ref, vne_ref, bond_ref, hin0_ref, tbl_ref):
    step = pl.program_id(0)
    iota128 = lax.broadcasted_iota(jnp.int32, (NB, 128), 1)
    acc = jnp.zeros((NB, D), jnp.float32)
    for f in range(9):
        col = x_ref[:, pl.ds(f, 1)]
        oh = (col == iota128).astype(jnp.float32)
        acc = acc + lax.dot(oh, atom_ref[f], preferred_element_type=jnp.float32)
    hin0_ref[...] = acc + vne_ref[...]

    @pl.when(step == 0)
    def _():
        b0 = bond_ref[0]
        b1 = bond_ref[1]
        b2 = bond_ref[2]
        b12 = (b1[:, None, :] + b2[None, :, :]).reshape(256, D)
        for i in range(16):
            tbl_ref[pl.ds(i * 256, 256), :] = b12 + b0[i, :][None, :]


def _fidx_body(ea_ref, fidx_ref):
    a = ea_ref[...]
    fidx_ref[...] = a[0] * 256 + a[1] * 16 + a[2]


def _layer_body(hin_ref, agg_ref, batch_ref, w1_ref, b1_ref, w2_ref, b2_ref,
                eps_ref, h_ref, *maybe_pooled, last):
    pooled_ref = maybe_pooled[0] if maybe_pooled else None
    step = pl.program_id(0)
    hin = hin_ref[...]
    pre = (1.0 + eps_ref[0, 0]) * hin + agg_ref[0] + agg_ref[1]
    t = lax.dot_general(pre, w1_ref[...], (((1,), (1,)), ((), ())),
                        preferred_element_type=jnp.float32)
    t = jnp.maximum((t + b1_ref[...]) * BN_SCALE, 0.0)
    h = lax.dot_general(t, w2_ref[...], (((1,), (1,)), ((), ())),
                        preferred_element_type=jnp.float32)
    h = (h + b2_ref[...]) * BN_SCALE
    if not last:
        h = jnp.maximum(h, 0.0)
    h_ref[...] = h
    if not last:
        oh = (batch_ref[...] == lax.broadcasted_iota(jnp.int32, (NB, G), 1)
              ).astype(jnp.float32)
        p = lax.dot_general(oh, hin, (((0,), (0,)), ((), ())),
                            preferred_element_type=jnp.float32)

        @pl.when(step == 0)
        def _():
            pooled_ref[...] = p

        @pl.when(step > 0)
        def _():
            pooled_ref[...] = pooled_ref[...] + p


def _vn_body(pooled_ref, vncur_ref, w1_ref, b1_ref, w2_ref, b2_ref,
             batch_ref, h_ref, vnnext_ref, hinnext_ref, vscr_ref):
    step = pl.program_id(0)

    @pl.when(step == 0)
    def _():
        pt = pooled_ref[...] + vncur_ref[...]
        t2 = lax.dot_general(pt, w1_ref[...], (((1,), (1,)), ((), ())),
                             preferred_element_type=jnp.float32)
        t2 = jnp.maximum((t2 + b1_ref[...]) * BN_SCALE, 0.0)
        v = lax.dot_general(t2, w2_ref[...], (((1,), (1,)), ((), ())),
                            preferred_element_type=jnp.float32)
        v = jnp.maximum((v + b2_ref[...]) * BN_SCALE, 0.0)
        vnnext_ref[...] = v
        vscr_ref[...] = v

    oh = (batch_ref[...] == lax.broadcasted_iota(jnp.int32, (NB, G), 1)
          ).astype(jnp.float32)
    hinnext_ref[...] = h_ref[...] + lax.dot(oh, vscr_ref[...],
                                            preferred_element_type=jnp.float32)


NC = 2
NS = 16
NW = NC * NS
EC = 128
CPW = 80
EPAD = NW * CPW * EC
NCH = EPAD // EC
NPAD = N + 8
RPT = 624


def _sc_agg_body(hin, tbl, gidx2, dst2, zeros, out,
                 big, gib, dib, sgm,
                 agg_sh):
    cid = lax.axis_index("c")
    sid = lax.axis_index("s")
    wid = sid * NC + cid

    lo = sid * RPT
    pltpu.sync_copy(zeros.at[pl.ds(lo, RPT)], agg_sh.at[pl.ds(lo, RPT)])

    @pl.when(sid == NS - 1)
    def _():
        pltpu.sync_copy(zeros.at[pl.ds(NS * RPT, NPAD - NS * RPT)],
                        agg_sh.at[pl.ds(NS * RPT, NPAD - NS * RPT)])

    plsc.subcore_barrier()

    def group(t, carry):
        g8 = (wid + NW * t) * 8
        pltpu.sync_copy(gidx2.at[pl.ds(2 * g8, 16)], gib)
        pltpu.sync_copy(dst2.at[pl.ds(g8, 8)], dib)

        for i in range(8):
            pltpu.async_copy(hin.at[gib.at[2 * i]],
                             big.at[pl.ds(0, EC)], sgm)
            pltpu.async_copy(tbl.at[gib.at[2 * i + 1]],
                             big.at[pl.ds(EC, EC)], sgm)
            pltpu.make_async_copy(hin.at[gib.at[2 * i]],
                                  big.at[pl.ds(0, EC)], sgm).wait()
            pltpu.make_async_copy(tbl.at[gib.at[2 * i + 1]],
                                  big.at[pl.ds(EC, EC)], sgm).wait()

            def erow(e, c2):
                for k in range(8):
                    s_ = pl.ds(k * 16, 16)
                    big[e, s_] = jnp.maximum(big[e, s_] + big[EC + e, s_], 0.0)
                return c2

            lax.fori_loop(0, EC, erow, 0, unroll=False)
            pltpu.sync_copy(big.at[pl.ds(0, EC)], agg_sh.at[dib.at[i]],
                            add=True)
        return carry

    lax.fori_loop(0, CPW // 8, group, 0, unroll=False)

    plsc.subcore_barrier()
    pltpu.sync_copy(agg_sh.at[pl.ds(lo, RPT)],
                    out.at[pl.ds(cid * N + lo, RPT)])

    @pl.when(sid == NS - 1)
    def _():
        pltpu.sync_copy(agg_sh.at[pl.ds(NS * RPT, N - NS * RPT)],
                        out.at[pl.ds(cid * N + NS * RPT, N - NS * RPT)])


def _sc_agg(hin, tbl, gidx2, dst2, zeros):
    mesh = plsc.VectorSubcoreMesh(core_axis_name="c", subcore_axis_name="s")
    f = pl.kernel(
        _sc_agg_body,
        out_type=jax.ShapeDtypeStruct((2 * N, D), jnp.float32),
        mesh=mesh,
        scratch_types=[
            pltpu.VMEM((2 * EC, D), jnp.float32),
            pltpu.VMEM((16, EC), jnp.int32),
            pltpu.VMEM((8, EC), jnp.int32),
            pltpu.SemaphoreType.DMA,
            pltpu.VMEM_SHARED((NPAD, D), jnp.float32),
        ],
    )
    return f(hin, tbl, gidx2, dst2, zeros)


def _prep(x, atom_emb, vn_emb, bond_emb):
    return pl.pallas_call(
        _prep_body,
        grid=(NSTEPS,),
        in_specs=[
            pl.BlockSpec((NB, 9), lambda i: (i, 0)),
            pl.BlockSpec((9, 128, D), lambda i: (0, 0, 0)),
            pl.BlockSpec((1, D), lambda i: (0, 0)),
            pl.BlockSpec((3, 16, D), lambda i: (0, 0, 0)),
        ],
        out_specs=[
            pl.BlockSpec((NB, D), lambda i: (i, 0)),
            pl.BlockSpec((4096, D), lambda i: (0, 0)),
        ],
        out_shape=[
            jax.ShapeDtypeStruct((N, D), jnp.float32),
            jax.ShapeDtypeStruct((4096, D), jnp.float32),
        ],
    )(x, atom_emb, vn_emb, bond_emb)


def _fidx(eat):
    return pl.pallas_call(
        _fidx_body,
        grid=(ER // EBR,),
        in_specs=[pl.BlockSpec((3, EBR, 128), lambda i: (0, i, 0))],
        out_specs=pl.BlockSpec((EBR, 128), lambda i: (i, 0)),
        out_shape=jax.ShapeDtypeStruct((ER, 128), jnp.int32),
    )(eat)


def _layer(hin, agg2, batch2, w1, b1, w2, b2, eps, last):
    out_specs = [pl.BlockSpec((NB, D), lambda i: (i, 0))]
    out_shape = [jax.ShapeDtypeStruct((N, D), jnp.float32)]
    if not last:
        out_specs.append(pl.BlockSpec((G, D), lambda i: (0, 0)))
        out_shape.append(jax.ShapeDtypeStruct((G, D), jnp.float32))
    res = pl.pallas_call(
        functools.partial(_layer_body, last=last),
        grid=(NSTEPS,),
        in_specs=[
            pl.BlockSpec((NB, D), lambda i: (i, 0)),
            pl.BlockSpec((2, NB, D), lambda i: (0, i, 0)),
            pl.BlockSpec((NB, 1), lambda i: (i, 0)),
            pl.BlockSpec((D, D), lambda i: (0, 0)),
            pl.BlockSpec((1, D), lambda i: (0, 0)),
            pl.BlockSpec((D, D), lambda i: (0, 0)),
            pl.BlockSpec((1, D), lambda i: (0, 0)),
            pl.BlockSpec(memory_space=pltpu.SMEM),
        ],
        out_specs=out_specs,
        out_shape=out_shape,
    )(hin, agg2, batch2, w1, b1, w2, b2, eps)
    if last:
        return res[0], None
    return res[0], res[1]


def _vn_apply(pooled, vncur, w1, b1, w2, b2, batch2, h):
    return pl.pallas_call(
        _vn_body,
        grid=(NSTEPS,),
        in_specs=[
            pl.BlockSpec((G, D), lambda i: (0, 0)),
            pl.BlockSpec((G, D), lambda i: (0, 0)),
            pl.BlockSpec((D, D), lambda i: (0, 0)),
            pl.BlockSpec((1, D), lambda i: (0, 0)),
            pl.BlockSpec((D, D), lambda i: (0, 0)),
            pl.BlockSpec((1, D), lambda i: (0, 0)),
            pl.BlockSpec((NB, 1), lambda i: (i, 0)),
            pl.BlockSpec((NB, D), lambda i: (i, 0)),
        ],
        out_specs=[
            pl.BlockSpec((G, D), lambda i: (0, 0)),
            pl.BlockSpec((NB, D), lambda i: (i, 0)),
        ],
        out_shape=[
            jax.ShapeDtypeStruct((G, D), jnp.float32),
            jax.ShapeDtypeStruct((N, D), jnp.float32),
        ],
        scratch_shapes=[pltpu.VMEM((G, D), jnp.float32)],
    )(pooled, vncur, w1, b1, w2, b2, batch2, h)


def kernel(x, edge_index, edge_attr, batch, atom_emb, bond_emb, vn_emb,
           gin_lin1_W, gin_lin1_b, gin_lin2_W, gin_lin2_b, gin_eps,
           vn_lin1_W, vn_lin1_b, vn_lin2_W, vn_lin2_b):
    pad = EPAD - E
    src1 = jnp.concatenate([edge_index[0], jnp.zeros((pad,), jnp.int32)])
    dst1 = jnp.concatenate([edge_index[1], jnp.full((pad,), N, jnp.int32)])
    eat = edge_attr.T.reshape(3, ER, 128)
    batch2 = batch.reshape(N, 1)
    zeros = jnp.zeros((NPAD, D), jnp.float32)

    hin0, tbl = _prep(x, atom_emb, vn_emb, bond_emb)
    fidx1 = jnp.concatenate([_fidx(eat).reshape(E),
                             jnp.zeros((pad,), jnp.int32)])
    gidx2 = jnp.concatenate([src1.reshape(NCH, 1, EC),
                             fidx1.reshape(NCH, 1, EC)],
                            axis=1).reshape(NCH * 2, EC)
    dst2 = dst1.reshape(NCH, EC)

    vn = jnp.broadcast_to(vn_emb, (G, D))
    hin = hin0
    h_list = [hin0]
    for layer in range(L):
        last = layer == L - 1
        agg2 = _sc_agg(hin, tbl, gidx2, dst2, zeros).reshape(2, N, D)
        eps = gin_eps[layer].reshape(1, 1)
        h, pooled = _layer(hin, agg2, batch2,
                           gin_lin1_W[layer], gin_lin1_b[layer].reshape(1, D),
                           gin_lin2_W[layer], gin_lin2_b[layer].reshape(1, D),
                           eps, last)
        if last:
            h_list.append(h)
        else:
            vn, hin = _vn_apply(pooled, vn,
                                vn_lin1_W[layer], vn_lin1_b[layer].reshape(1, D),
                                vn_lin2_W[layer], vn_lin2_b[layer].reshape(1, D),
                                batch2, h)
            h_list.append(hin)
    return jnp.stack(h_list)


# speedup vs baseline: 1.4630x; 1.2558x over previous
import functools

import jax
import jax.numpy as jnp
from jax import lax
from jax.experimental import pallas as pl
from jax.experimental.pallas import tpu as pltpu
from jax.experimental.pallas import tpu_sc as plsc

N = 10000
E = 320000
D = 128
G = 128
L = 3
BN_SCALE = float(1.0 / (1.0 + 1e-5) ** 0.5)


NB = 2000
NSTEPS = N // NB

ER = E // 128
EBR = ER


def _prep_body(x_ref, atom_ref, vne_ref, bond_ref, hin0_ref, tbl_ref):
    step = pl.program_id(0)
    iota128 = lax.broadcasted_iota(jnp.int32, (NB, 128), 1)
    acc = jnp.zeros((NB, D), jnp.float32)
    for f in range(9):
        col = x_ref[:, pl.ds(f, 1)]
        oh = (col == iota128).astype(jnp.float32)
        acc = acc + lax.dot(oh, atom_ref[f], preferred_element_type=jnp.float32)
    hin0_ref[...] = acc + vne_ref[...]

    @pl.when(step == 0)
    def _():
        b0 = bond_ref[0]
        b1 = bond_ref[1]
        b2 = bond_ref[2]
        b12 = (b1[:, None, :] + b2[None, :, :]).reshape(256, D)
        for i in range(16):
            tbl_ref[pl.ds(i * 256, 256), :] = b12 + b0[i, :][None, :]


def _fidx_body(ea_ref, fidx_ref):
    a = ea_ref[...]
    fidx_ref[...] = a[0] * 256 + a[1] * 16 + a[2]


def _layer_body(hin_ref, agg_ref, batch_ref, w1_ref, b1_ref, w2_ref, b2_ref,
                eps_ref, h_ref, *maybe_pooled, last):
    pooled_ref = maybe_pooled[0] if maybe_pooled else None
    step = pl.program_id(0)
    hin = hin_ref[...]
    pre = (1.0 + eps_ref[0, 0]) * hin + agg_ref[0] + agg_ref[1]
    t = lax.dot_general(pre, w1_ref[...], (((1,), (1,)), ((), ())),
                        preferred_element_type=jnp.float32)
    t = jnp.maximum((t + b1_ref[...]) * BN_SCALE, 0.0)
    h = lax.dot_general(t, w2_ref[...], (((1,), (1,)), ((), ())),
                        preferred_element_type=jnp.float32)
    h = (h + b2_ref[...]) * BN_SCALE
    if not last:
        h = jnp.maximum(h, 0.0)
    h_ref[...] = h
    if not last:
        oh = (batch_ref[...] == lax.broadcasted_iota(jnp.int32, (NB, G), 1)
              ).astype(jnp.float32)
        p = lax.dot_general(oh, hin, (((0,), (0,)), ((), ())),
                            preferred_element_type=jnp.float32)

        @pl.when(step == 0)
        def _():
            pooled_ref[...] = p

        @pl.when(step > 0)
        def _():
            pooled_ref[...] = pooled_ref[...] + p


def _vn_body(pooled_ref, vncur_ref, w1_ref, b1_ref, w2_ref, b2_ref,
             batch_ref, h_ref, vnnext_ref, hinnext_ref, vscr_ref):
    step = pl.program_id(0)

    @pl.when(step == 0)
    def _():
        pt = pooled_ref[...] + vncur_ref[...]
        t2 = lax.dot_general(pt, w1_ref[...], (((1,), (1,)), ((), ())),
                             preferred_element_type=jnp.float32)
        t2 = jnp.maximum((t2 + b1_ref[...]) * BN_SCALE, 0.0)
        v = lax.dot_general(t2, w2_ref[...], (((1,), (1,)), ((), ())),
                            preferred_element_type=jnp.float32)
        v = jnp.maximum((v + b2_ref[...]) * BN_SCALE, 0.0)
        vnnext_ref[...] = v
        vscr_ref[...] = v

    oh = (batch_ref[...] == lax.broadcasted_iota(jnp.int32, (NB, G), 1)
          ).astype(jnp.float32)
    hinnext_ref[...] = h_ref[...] + lax.dot(oh, vscr_ref[...],
                                            preferred_element_type=jnp.float32)


NC = 2
NS = 16
NW = NC * NS
EC = 128
NCHUNK = E // EC
RPT = 624
RREM = N - NS * RPT


def _sc_agg_body(hin, tbl, src, dst, fidx, zeros, out,
                 src_v, dst_v, fidx_v, msg_v, ea_v, agg_sh, sem, sem2):
    cid = lax.axis_index("c")
    sid = lax.axis_index("s")
    wid = sid * NC + cid

    lo = sid * RPT
    pltpu.sync_copy(zeros.at[pl.ds(lo, RPT)], agg_sh.at[pl.ds(lo, RPT)])

    @pl.when(sid == NS - 1)
    def _():
        pltpu.sync_copy(zeros.at[pl.ds(NS * RPT, RREM)],
                        agg_sh.at[pl.ds(NS * RPT, RREM)])

    plsc.subcore_barrier()

    nchunks = 78 + jnp.where(wid < NCHUNK - 78 * NW, 1, 0)

    def chunk(i, carry):
        off = (wid + i * NW) * EC
        pltpu.sync_copy(src.at[pl.ds(off, EC)], src_v)
        pltpu.sync_copy(fidx.at[pl.ds(off, EC)], fidx_v)
        pltpu.sync_copy(dst.at[pl.ds(off, EC)], dst_v)
        pltpu.async_copy(hin.at[src_v], msg_v, sem).wait()
        pltpu.async_copy(tbl.at[fidx_v], ea_v, sem2).wait()

        def erow(e, c2):
            for k in range(8):
                s_ = pl.ds(k * 16, 16)
                msg_v[e, s_] = jnp.maximum(msg_v[e, s_] + ea_v[e, s_], 0.0)
            return c2

        lax.fori_loop(0, EC, erow, 0, unroll=False)
        pltpu.sync_copy(msg_v, agg_sh.at[dst_v], add=True)
        return carry

    lax.fori_loop(0, nchunks, chunk, 0, unroll=False)
    plsc.subcore_barrier()
    pltpu.sync_copy(agg_sh.at[pl.ds(lo, RPT)],
                    out.at[pl.ds(cid * N + lo, RPT)])

    @pl.when(sid == NS - 1)
    def _():
        pltpu.sync_copy(agg_sh.at[pl.ds(NS * RPT, RREM)],
                        out.at[pl.ds(cid * N + NS * RPT, RREM)])


def _sc_agg(hin, tbl, src, dst, fidx, zeros):
    mesh = plsc.VectorSubcoreMesh(core_axis_name="c", subcore_axis_name="s")
    f = pl.kernel(
        _sc_agg_body,
        out_type=jax.ShapeDtypeStruct((2 * N, D), jnp.float32),
        mesh=mesh,
        scratch_types=[
            pltpu.VMEM((EC,), jnp.int32),
            pltpu.VMEM((EC,), jnp.int32),
            pltpu.VMEM((EC,), jnp.int32),
            pltpu.VMEM((EC, D), jnp.float32),
            pltpu.VMEM((EC, D), jnp.float32),
            pltpu.VMEM_SHARED((N, D), jnp.float32),
            pltpu.SemaphoreType.DMA,
            pltpu.SemaphoreType.DMA,
        ],
    )
    return f(hin, tbl, src, dst, fidx, zeros)


def _prep(x, atom_emb, vn_emb, bond_emb):
    return pl.pallas_call(
        _prep_body,
        grid=(NSTEPS,),
        in_specs=[
            pl.BlockSpec((NB, 9), lambda i: (i, 0)),
            pl.BlockSpec((9, 128, D), lambda i: (0, 0, 0)),
            pl.BlockSpec((1, D), lambda i: (0, 0)),
            pl.BlockSpec((3, 16, D), lambda i: (0, 0, 0)),
        ],
        out_specs=[
            pl.BlockSpec((NB, D), lambda i: (i, 0)),
            pl.BlockSpec((4096, D), lambda i: (0, 0)),
        ],
        out_shape=[
            jax.ShapeDtypeStruct((N, D), jnp.float32),
            jax.ShapeDtypeStruct((4096, D), jnp.float32),
        ],
    )(x, atom_emb, vn_emb, bond_emb)


def _fidx(eat):
    return pl.pallas_call(
        _fidx_body,
        grid=(ER // EBR,),
        in_specs=[pl.BlockSpec((3, EBR, 128), lambda i: (0, i, 0))],
        out_specs=pl.BlockSpec((EBR, 128), lambda i: (i, 0)),
        out_shape=jax.ShapeDtypeStruct((ER, 128), jnp.int32),
    )(eat)


def _layer(hin, agg2, batch2, w1, b1, w2, b2, eps, last):
    out_specs = [pl.BlockSpec((NB, D), lambda i: (i, 0))]
    out_shape = [jax.ShapeDtypeStruct((N, D), jnp.float32)]
    if not last:
        out_specs.append(pl.BlockSpec((G, D), lambda i: (0, 0)))
        out_shape.append(jax.ShapeDtypeStruct((G, D), jnp.float32))
    res = pl.pallas_call(
        functools.partial(_layer_body, last=last),
        grid=(NSTEPS,),
        in_specs=[
            pl.BlockSpec((NB, D), lambda i: (i, 0)),
            pl.BlockSpec((2, NB, D), lambda i: (0, i, 0)),
            pl.BlockSpec((NB, 1), lambda i: (i, 0)),
            pl.BlockSpec((D, D), lambda i: (0, 0)),
            pl.BlockSpec((1, D), lambda i: (0, 0)),
            pl.BlockSpec((D, D), lambda i: (0, 0)),
            pl.BlockSpec((1, D), lambda i: (0, 0)),
            pl.BlockSpec(memory_space=pltpu.SMEM),
        ],
        out_specs=out_specs,
        out_shape=out_shape,
    )(hin, agg2, batch2, w1, b1, w2, b2, eps)
    if last:
        return res[0], None
    return res[0], res[1]


def _vn_apply(pooled, vncur, w1, b1, w2, b2, batch2, h):
    return pl.pallas_call(
        _vn_body,
        grid=(NSTEPS,),
        in_specs=[
            pl.BlockSpec((G, D), lambda i: (0, 0)),
            pl.BlockSpec((G, D), lambda i: (0, 0)),
            pl.BlockSpec((D, D), lambda i: (0, 0)),
            pl.BlockSpec((1, D), lambda i: (0, 0)),
            pl.BlockSpec((D, D), lambda i: (0, 0)),
            pl.BlockSpec((1, D), lambda i: (0, 0)),
            pl.BlockSpec((NB, 1), lambda i: (i, 0)),
            pl.BlockSpec((NB, D), lambda i: (i, 0)),
        ],
        out_specs=[
            pl.BlockSpec((G, D), lambda i: (0, 0)),
            pl.BlockSpec((NB, D), lambda i: (i, 0)),
        ],
        out_shape=[
            jax.ShapeDtypeStruct((G, D), jnp.float32),
            jax.ShapeDtypeStruct((N, D), jnp.float32),
        ],
        scratch_shapes=[pltpu.VMEM((G, D), jnp.float32)],
    )(pooled, vncur, w1, b1, w2, b2, batch2, h)


def kernel(x, edge_index, edge_attr, batch, atom_emb, bond_emb, vn_emb,
           gin_lin1_W, gin_lin1_b, gin_lin2_W, gin_lin2_b, gin_eps,
           vn_lin1_W, vn_lin1_b, vn_lin2_W, vn_lin2_b):
    src = edge_index[0]
    dst = edge_index[1]
    eat = edge_attr.T.reshape(3, ER, 128)
    batch2 = batch.reshape(N, 1)
    zeros = jnp.zeros((N, D), jnp.float32)

    hin0, tbl = _prep(x, atom_emb, vn_emb, bond_emb)
    fidx = _fidx(eat).reshape(E)

    vn = jnp.broadcast_to(vn_emb, (G, D))
    hin = hin0
    h_list = [hin0]
    for layer in range(L):
        last = layer == L - 1
        agg2 = _sc_agg(hin, tbl, src, dst, fidx, zeros).reshape(2, N, D)
        eps = gin_eps[layer].reshape(1, 1)
        h, pooled = _layer(hin, agg2, batch2,
                           gin_lin1_W[layer], gin_lin1_b[layer].reshape(1, D),
                           gin_lin2_W[layer], gin_lin2_b[layer].reshape(1, D),
                           eps, last)
        if last:
            h_list.append(h)
        else:
            vn, hin = _vn_apply(pooled, vn,
                                vn_lin1_W[layer], vn_lin1_b[layer].reshape(1, D),
                                vn_lin2_W[layer], vn_lin2_b[layer].reshape(1, D),
                                batch2, h)
            h_list.append(hin)
    return jnp.stack(h_list)
